# Initial kernel scaffold; baseline (speedup 1.0000x reference)
import jax
import jax.numpy as jnp
from jax.experimental import pallas as pl


def kernel(in_nodes_features, edge_index, Wp, a_src, a_trg, Wskip):
    raise NotImplementedError("write your pallas kernel here")


# speedup vs baseline: 4.7903x; 4.7903x over previous
import functools

import jax
import jax.numpy as jnp
from jax import lax
from jax.experimental import pallas as pl
from jax.experimental.pallas import tpu as pltpu
from jax.experimental.pallas import tpu_sc as plsc

N_NODES = 1000
D_IN = 64
N_HEADS = 4
F_OUT = 16
HF = N_HEADS * F_OUT
NG = 2
GF = HF // NG
N_EDGES = 16000
LANES = 16
SC_CORES = 2
SC_SUBCORES = 16
N_WORKERS = SC_CORES * SC_SUBCORES


def _pre_body(x_ref, wp_ref, acat_ref, proj_ref, scores_ref):
    x = x_ref[0]
    p = jnp.dot(x, wp_ref[...], preferred_element_type=jnp.float32)
    proj_ref[0, 0] = p[:, :GF]
    proj_ref[0, 1] = p[:, GF:]
    for g in range(NG):
        scores_ref[0, g] = jnp.dot(
            p, acat_ref[g], preferred_element_type=jnp.float32)


def _tc_pre(x, wp, acat, bt):
    return pl.pallas_call(
        _pre_body,
        grid=(bt,),
        in_specs=[
            pl.BlockSpec((1, N_NODES, D_IN), lambda i: (i, 0, 0)),
            pl.BlockSpec((D_IN, HF), lambda i: (0, 0)),
            pl.BlockSpec((NG, HF, 4), lambda i: (0, 0, 0)),
        ],
        out_specs=[
            pl.BlockSpec((1, NG, N_NODES, GF), lambda i: (i, 0, 0, 0)),
            pl.BlockSpec((1, NG, N_NODES, 4), lambda i: (i, 0, 0, 0)),
        ],
        out_shape=[
            jax.ShapeDtypeStruct((bt, NG, N_NODES, GF), jnp.float32),
            jax.ShapeDtypeStruct((bt, NG, N_NODES, 4), jnp.float32),
        ],
    )(x, wp, acat)


def _sc_edge(proj, scores, src, trg, bt):
    units = bt * NG
    units_per_worker = units // N_WORKERS
    mesh = plsc.VectorSubcoreMesh(
        core_axis_name="c", subcore_axis_name="s",
        num_cores=SC_CORES, num_subcores=SC_SUBCORES)
    n_chunks = N_EDGES // LANES

    @functools.partial(
        pl.kernel,
        out_type=(
            jax.ShapeDtypeStruct((bt, NG, N_NODES * GF), jnp.float32),
            jax.ShapeDtypeStruct((bt, NG, N_NODES * NG), jnp.float32),
        ),
        mesh=mesh,
        compiler_params=pltpu.CompilerParams(needs_layout_passes=False),
        scratch_types=[
            pltpu.VMEM((N_EDGES,), jnp.int32),
            pltpu.VMEM((N_EDGES,), jnp.int32),
            pltpu.VMEM((N_NODES * GF,), jnp.float32),
            pltpu.VMEM((N_NODES * 4,), jnp.float32),
            pltpu.VMEM((N_NODES * GF,), jnp.float32),
            pltpu.VMEM((N_NODES * NG,), jnp.float32),
        ],
    )
    def body(proj_hbm, scores_hbm, src_hbm, trg_hbm, acc_hbm, den_hbm,
             src_v, trg_v, proj_v, scores_v, acc_v, den_v):
        wid = lax.axis_index("s") * SC_CORES + lax.axis_index("c")
        pltpu.sync_copy(src_hbm, src_v)
        pltpu.sync_copy(trg_hbm, trg_v)

        zeros16 = jnp.zeros((LANES,), jnp.float32)

        for u in range(units_per_worker):
            unit = wid * units_per_worker + u
            ibt = unit // NG
            g = unit % NG
            pltpu.sync_copy(proj_hbm.at[ibt, g], proj_v)
            pltpu.sync_copy(scores_hbm.at[ibt, g], scores_v)

            def zero_acc(k, _):
                acc_v[pl.ds(k * LANES, LANES)] = zeros16
                return 0
            lax.fori_loop(0, N_NODES * GF // LANES, zero_acc, 0)

            def zero_den(k, _):
                den_v[pl.ds(k * LANES, LANES)] = zeros16
                return 0
            lax.fori_loop(0, N_NODES * NG // LANES, zero_den, 0)

            def edge_chunk(i, _):
                s16 = src_v[pl.ds(i * LANES, LANES)]
                t16 = trg_v[pl.ds(i * LANES, LANES)]
                sb = s16 * 4
                tb = t16 * 4
                es = []
                for hh in range(NG):
                    ss = plsc.load_gather(scores_v, [sb + hh])
                    st = plsc.load_gather(scores_v, [tb + 2 + hh])
                    sc = ss + st
                    e = jnp.exp(jnp.maximum(sc, 0.2 * sc))
                    es.append(e)
                    plsc.addupdate_scatter(den_v, [t16 * NG + hh], e)
                pb = s16 * GF
                ab = t16 * GF
                for j in range(GF):
                    p = plsc.load_gather(proj_v, [pb + j])
                    plsc.addupdate_scatter(acc_v, [ab + j], p * es[j // F_OUT])
                return 0
            lax.fori_loop(0, n_chunks, edge_chunk, 0)

            pltpu.sync_copy(acc_v, acc_hbm.at[ibt, g])
            pltpu.sync_copy(den_v, den_hbm.at[ibt, g])

    return body(proj.reshape(bt, NG, N_NODES * GF),
                scores.reshape(bt, NG, N_NODES * 4),
                src, trg)


def _post_body(x_ref, acc_ref, den_ref, wskip_ref, p2_ref, out_ref):
    sk = jnp.dot(x_ref[0], wskip_ref[...], preferred_element_type=jnp.float32)
    for g in range(NG):
        r = 1.0 / (den_ref[0, g] + 1e-16)
        rexp = jnp.dot(r, p2_ref[...], preferred_element_type=jnp.float32)
        og = acc_ref[0, g] * rexp + sk[:, g * GF:(g + 1) * GF]
        out_ref[0, :, g * GF:(g + 1) * GF] = jnp.where(
            og > 0, og, jnp.exp(jnp.minimum(og, 0.0)) - 1.0)


def _tc_post(x, acc, den, wskip, p2, bt):
    return pl.pallas_call(
        _post_body,
        grid=(bt,),
        in_specs=[
            pl.BlockSpec((1, N_NODES, D_IN), lambda i: (i, 0, 0)),
            pl.BlockSpec((1, NG, N_NODES, GF), lambda i: (i, 0, 0, 0)),
            pl.BlockSpec((1, NG, N_NODES, NG), lambda i: (i, 0, 0, 0)),
            pl.BlockSpec((D_IN, HF), lambda i: (0, 0)),
            pl.BlockSpec((NG, GF), lambda i: (0, 0)),
        ],
        out_specs=pl.BlockSpec((1, N_NODES, HF), lambda i: (i, 0, 0)),
        out_shape=jax.ShapeDtypeStruct((bt, N_NODES, HF), jnp.float32),
    )(x, acc, den, wskip, p2)


def kernel(in_nodes_features, edge_index, Wp, a_src, a_trg, Wskip):
    B, T, N, Din = in_nodes_features.shape
    H, F = a_src.shape
    bt = B * T
    x = in_nodes_features.reshape(bt, N, Din)
    src = edge_index[0].astype(jnp.int32)
    trg = edge_index[1].astype(jnp.int32)

    eyeH = jnp.eye(H, dtype=jnp.float32)
    a_s = (a_src[:, None, :] * eyeH[:, :, None]).transpose(0, 2, 1).reshape(HF, H)
    a_t = (a_trg[:, None, :] * eyeH[:, :, None]).transpose(0, 2, 1).reshape(HF, H)
    acat = jnp.stack([
        jnp.concatenate(
            [a_s[:, g * 2:g * 2 + 2], a_t[:, g * 2:g * 2 + 2]], axis=1)
        for g in range(NG)], axis=0)
    p2 = jnp.repeat(jnp.eye(NG, dtype=jnp.float32), F_OUT, axis=1)

    proj, scores = _tc_pre(x, Wp, acat, bt)
    acc, den = _sc_edge(proj, scores, src, trg, bt)
    acc = acc.reshape(bt, NG, N, GF)
    den = den.reshape(bt, NG, N, NG)
    out = _tc_post(x, acc, den, Wskip, p2, bt)
    return out.reshape(B, T, N, HF)


# speedup vs baseline: 23.7919x; 4.9667x over previous
import functools

import jax
import jax.numpy as jnp
from jax import lax
from jax.experimental import pallas as pl
from jax.experimental.pallas import tpu as pltpu
from jax.experimental.pallas import tpu_sc as plsc

N_NODES = 1000
D_IN = 64
N_HEADS = 4
F_OUT = 16
HF = N_HEADS * F_OUT
NG = 2
GF = HF // NG
N_EDGES = 16000
EDGE_BLK = 4000
LANES = 16
SC_CORES = 2
SC_SUBCORES = 16
N_WORKERS = SC_CORES * SC_SUBCORES
NP = 1024


def _pre_body(x_ref, wcat_ref, proj_ref):
    p = jnp.dot(x_ref[0], wcat_ref[...], preferred_element_type=jnp.float32)
    proj_ref[0, :N_NODES] = p
    proj_ref[0, N_NODES:] = jnp.zeros((NP - N_NODES, 128), jnp.float32)


def _tc_pre(x, wcat, bt):
    return pl.pallas_call(
        _pre_body,
        grid=(bt,),
        in_specs=[
            pl.BlockSpec((1, N_NODES, D_IN), lambda i: (i, 0, 0)),
            pl.BlockSpec((D_IN, 128), lambda i: (0, 0)),
        ],
        out_specs=pl.BlockSpec((1, NP, 128), lambda i: (i, 0, 0)),
        out_shape=jax.ShapeDtypeStruct((bt, NP, 128), jnp.float32),
    )(x, wcat)


def _sc_edge(proj, avec, src, trg, bt):
    units = bt * NG
    units_per_worker = units // N_WORKERS
    mesh = plsc.VectorSubcoreMesh(
        core_axis_name="c", subcore_axis_name="s",
        num_cores=SC_CORES, num_subcores=SC_SUBCORES)

    @functools.partial(
        pl.kernel,
        out_type=jax.ShapeDtypeStruct((bt, NP, 128), jnp.float32),
        mesh=mesh,
        compiler_params=pltpu.CompilerParams(
            needs_layout_passes=False, use_tc_tiling_on_sc=False),
        scratch_types=[
            pltpu.VMEM((N_EDGES,), jnp.int32),
            pltpu.VMEM((N_EDGES,), jnp.int32),
            pltpu.VMEM((NP, GF), jnp.float32),
            pltpu.VMEM((NP * 4,), jnp.float32),
            pltpu.VMEM((NP, GF), jnp.float32),
            pltpu.VMEM((NP * NG,), jnp.float32),
            pltpu.VMEM((EDGE_BLK,), jnp.float32),
            pltpu.VMEM((EDGE_BLK,), jnp.float32),
            pltpu.VMEM((128,), jnp.float32),
        ],
    )
    def body(proj_hbm, avec_hbm, src_hbm, trg_hbm, acc_hbm,
             src_v, trg_v, proj_v, scores_v, acc_v, den_v, esc0_v, esc1_v,
             avec_v):
        wid = lax.axis_index("s") * SC_CORES + lax.axis_index("c")
        pltpu.sync_copy(src_hbm, src_v)
        pltpu.sync_copy(trg_hbm, trg_v)
        pltpu.sync_copy(avec_hbm, avec_v)

        zeros16 = jnp.zeros((LANES,), jnp.float32)
        iota16 = jax.lax.iota(jnp.int32, LANES)

        for u in range(units_per_worker):
            unit = wid * units_per_worker + u
            ibt = unit // NG
            g = unit % NG
            pltpu.sync_copy(proj_hbm.at[ibt, :, pl.ds(g * GF, GF)], proj_v)

            av = [avec_v[pl.ds(g * 64 + q * LANES, LANES)] for q in range(4)]

            def zero_acc(r, _):
                acc_v[r, pl.ds(0, LANES)] = zeros16
                acc_v[r, pl.ds(LANES, LANES)] = zeros16
                return 0
            lax.fori_loop(0, NP, zero_acc, 0)

            def zero_den(k, _):
                den_v[pl.ds(k * LANES, LANES)] = zeros16
                return 0
            lax.fori_loop(0, NP * NG // LANES, zero_den, 0)

            a0s = [av[0][j] for j in range(LANES)]
            a1s = [av[1][j] for j in range(LANES)]
            a2s = [av[2][j] for j in range(LANES)]
            a3s = [av[3][j] for j in range(LANES)]

            def score_node16(c, _):
                nvec = iota16 + c * LANES
                s0 = zeros16
                s1 = zeros16
                s2 = zeros16
                s3 = zeros16
                for j in range(LANES):
                    jv = jnp.full((LANES,), j, jnp.int32)
                    pj = plsc.load_gather(proj_v, [nvec, jv])
                    s0 = s0 + pj * a0s[j]
                    s2 = s2 + pj * a2s[j]
                for j in range(LANES):
                    jv = jnp.full((LANES,), LANES + j, jnp.int32)
                    pj = plsc.load_gather(proj_v, [nvec, jv])
                    s1 = s1 + pj * a1s[j]
                    s3 = s3 + pj * a3s[j]
                sbase = nvec * 4
                plsc.store_scatter(scores_v, [sbase], s0)
                plsc.store_scatter(scores_v, [sbase + 1], s1)
                plsc.store_scatter(scores_v, [sbase + 2], s2)
                plsc.store_scatter(scores_v, [sbase + 3], s3)
                return 0
            lax.fori_loop(0, NP // LANES, score_node16, 0)

            for blk in range(N_EDGES // EDGE_BLK):
                eb = blk * EDGE_BLK

                def score_chunk(c, _):
                    s16 = src_v[pl.ds(eb + c * LANES, LANES)]
                    t16 = trg_v[pl.ds(eb + c * LANES, LANES)]
                    sb = s16 * 4
                    tb = t16 * 4
                    for hh in range(NG):
                        ss = plsc.load_gather(scores_v, [sb + hh])
                        st = plsc.load_gather(scores_v, [tb + 2 + hh])
                        sc = ss + st
                        e = jnp.exp(jnp.maximum(sc, 0.2 * sc))
                        (esc0_v if hh == 0 else esc1_v)[
                            pl.ds(c * LANES, LANES)] = e
                        plsc.addupdate_scatter(den_v, [t16 * NG + hh], e)
                    return 0
                lax.fori_loop(0, EDGE_BLK // LANES, score_chunk, 0)

                def feat_chunk(c, _):
                    s16 = src_v[pl.ds(eb + c * LANES, LANES)]
                    t16 = trg_v[pl.ds(eb + c * LANES, LANES)]
                    e0v = esc0_v[pl.ds(c * LANES, LANES)]
                    e1v = esc1_v[pl.ds(c * LANES, LANES)]
                    for half in range(2):
                        loads = []
                        for k in range(half * 8, half * 8 + 8):
                            s = s16[k]
                            p0 = proj_v[s, pl.ds(0, LANES)]
                            p1 = proj_v[s, pl.ds(LANES, LANES)]
                            loads.append((t16[k], p0 * e0v[k], p1 * e1v[k]))
                        for t, w0, w1 in loads:
                            plsc.addupdate(acc_v.at[t, pl.ds(0, LANES)], w0)
                            plsc.addupdate(
                                acc_v.at[t, pl.ds(LANES, LANES)], w1)
                    return 0
                lax.fori_loop(0, EDGE_BLK // LANES, feat_chunk, 0)

            def recip_chunk(k, _):
                d = den_v[pl.ds(k * LANES, LANES)]
                den_v[pl.ds(k * LANES, LANES)] = 1.0 / (d + 1e-16)
                return 0
            lax.fori_loop(0, NP * NG // LANES, recip_chunk, 0)

            def norm_chunk(q, _):
                dvec = den_v[pl.ds(q * LANES, LANES)]
                for k in range(8):
                    r = q * 8 + k
                    a0 = acc_v[r, pl.ds(0, LANES)]
                    a1 = acc_v[r, pl.ds(LANES, LANES)]
                    acc_v[r, pl.ds(0, LANES)] = a0 * dvec[2 * k]
                    acc_v[r, pl.ds(LANES, LANES)] = a1 * dvec[2 * k + 1]
                return 0
            lax.fori_loop(0, NP // 8, norm_chunk, 0)

            pltpu.sync_copy(acc_v, acc_hbm.at[ibt, :, pl.ds(g * GF, GF)])

    return body(proj, avec, src, trg)


def _post_body(x_ref, acc_ref, wskip_ref, out_ref):
    sk = jnp.dot(x_ref[0], wskip_ref[...], preferred_element_type=jnp.float32)
    og = acc_ref[0, :N_NODES, :HF] + sk
    out_ref[0] = jnp.where(og > 0, og, jnp.exp(jnp.minimum(og, 0.0)) - 1.0)


def _tc_post(x, acc, wskip, bt):
    return pl.pallas_call(
        _post_body,
        grid=(bt,),
        in_specs=[
            pl.BlockSpec((1, N_NODES, D_IN), lambda i: (i, 0, 0)),
            pl.BlockSpec((1, NP, 128), lambda i: (i, 0, 0)),
            pl.BlockSpec((D_IN, HF), lambda i: (0, 0)),
        ],
        out_specs=pl.BlockSpec((1, N_NODES, HF), lambda i: (i, 0, 0)),
        out_shape=jax.ShapeDtypeStruct((bt, N_NODES, HF), jnp.float32),
    )(x, acc, wskip)


def kernel(in_nodes_features, edge_index, Wp, a_src, a_trg, Wskip):
    B, T, N, Din = in_nodes_features.shape
    bt = B * T
    x = in_nodes_features.reshape(bt, N, Din)
    src = edge_index[0].astype(jnp.int32)
    trg = edge_index[1].astype(jnp.int32)

    wcat = jnp.pad(Wp, ((0, 0), (0, 128 - HF)))
    avec = jnp.concatenate(
        [jnp.concatenate([a_src[2 * g], a_src[2 * g + 1],
                          a_trg[2 * g], a_trg[2 * g + 1]])
         for g in range(NG)])

    proj = _tc_pre(x, wcat, bt)
    acc = _sc_edge(proj, avec, src, trg, bt)
    out = _tc_post(x, acc, Wskip, bt)
    return out.reshape(B, T, N, HF)
